# baseline (speedup 1.0000x reference)
import functools

import jax
import jax.numpy as jnp
from jax import lax
from jax.experimental import pallas as pl
from jax.experimental.pallas import tpu as pltpu
from jax.experimental.pallas import tpu_sc as plsc

NBUF = 4
CW = 128


def _build(B, T, D, vocab):
    info = plsc.get_sparse_core_info()
    NC, NS = info.num_cores, info.num_subcores
    NW = NC * NS
    BT = B * T
    assert BT % (NW * CW) == 0
    rows_w = BT // NW
    n_chunks = rows_w // CW
    assert rows_w % T == 0
    assert (CW % 8 == 0) and (T % 8 == 0)
    assert n_chunks % NBUF == 0
    PT = T + CW + 8

    mesh = plsc.VectorSubcoreMesh(core_axis_name="c", subcore_axis_name="s")

    @functools.partial(
        pl.kernel,
        out_type=jax.ShapeDtypeStruct((BT, D), jnp.float32),
        mesh=mesh,
        compiler_params=pltpu.CompilerParams(use_tc_tiling_on_sc=False),
        scratch_types=(
            [pltpu.VMEM((n_chunks, CW), jnp.int32)]
            + [pltpu.VMEM_SHARED((PT, D), jnp.float32)]
            + [pltpu.VMEM((CW, D), jnp.float32)] * NBUF
            + [pltpu.SemaphoreType.DMA] * (3 * NBUF)
        ),
    )
    def enc(obs_hbm, embed_hbm, pos_hbm, out_hbm, idx_v, pos_sh, *rest):
        bufs = rest[:NBUF]
        ssem = rest[NBUF:2 * NBUF]
        gsem = rest[2 * NBUF:3 * NBUF]
        osem = rest[3 * NBUF:4 * NBUF]
        sid = lax.axis_index("s")
        wid = sid * NC + lax.axis_index("c")
        base = wid * rows_w
        pltpu.sync_copy(obs_hbm.at[wid], idx_v)
        @pl.when(sid == 0)
        def _():
            pltpu.sync_copy(pos_hbm, pos_sh)
        plsc.subcore_barrier()

        def seed(s, b):
            off = (s * CW) % T
            pltpu.async_copy(pos_sh.at[pl.ds(off, CW)], bufs[b], ssem[b])

        seed(0, 0)
        seed(1, 1)

        def visit(s, b):
            pltpu.make_async_copy(pos_sh.at[pl.ds(0, CW)], bufs[b],
                                  ssem[b]).wait()
            d0 = pltpu.async_copy(embed_hbm.at[idx_v.at[s]], bufs[b],
                                  gsem[b], add=True)
            b2 = (b + 2) % NBUF
            @pl.when(s + 2 < n_chunks)
            def _():
                @pl.when(s >= 2)
                def _():
                    pltpu.make_async_copy(
                        bufs[b2], out_hbm.at[pl.ds(base, CW)],
                        osem[b2]).wait()
                seed(s + 2, b2)
            d0.wait()
            pltpu.async_copy(bufs[b], out_hbm.at[pl.ds(base + s * CW, CW)],
                             osem[b])

        def body(g, carry):
            for b in range(NBUF):
                visit(NBUF * g + b, b)
            return carry

        lax.fori_loop(0, n_chunks // NBUF, body, 0)
        for b in range(NBUF):
            pltpu.make_async_copy(bufs[b], out_hbm.at[pl.ds(base, CW)],
                                  osem[b]).wait()

    return enc


def kernel(obs, embed, pos_embed):
    B, T = obs.shape
    vocab, D = embed.shape
    enc = _build(B, T, D, vocab)
    info = plsc.get_sparse_core_info()
    NW = info.num_cores * info.num_subcores
    obs_i = obs.astype(jnp.int32).reshape(NW, (B * T) // (NW * CW), CW)
    pos_t = pos_embed[:T]
    pos_w = jnp.concatenate([pos_t, pos_t[:CW + 8]], axis=0)
    out = enc(obs_i, embed, pos_w)
    return out.reshape(B, T, D)


# speedup vs baseline: 1.6216x; 1.6216x over previous
import functools

import jax
import jax.numpy as jnp
from jax import lax
from jax.experimental import pallas as pl
from jax.experimental.pallas import tpu as pltpu
from jax.experimental.pallas import tpu_sc as plsc

NBUF = 4
CW = 128


def _build(B, T, D, vocab):
    info = plsc.get_sparse_core_info()
    NC, NS = info.num_cores, info.num_subcores
    NW = NC * NS
    BT = B * T
    assert BT % (NW * CW) == 0
    rows_w = BT // NW
    n_chunks = rows_w // CW
    assert rows_w % T == 0
    assert (CW % 8 == 0) and (T % 8 == 0)
    assert n_chunks % NBUF == 0
    PT = T + CW + 8

    mesh = plsc.VectorSubcoreMesh(core_axis_name="c", subcore_axis_name="s")

    @functools.partial(
        pl.kernel,
        out_type=jax.ShapeDtypeStruct((BT, 128), jnp.float32),
        mesh=mesh,
        compiler_params=pltpu.CompilerParams(use_tc_tiling_on_sc=False),
        scratch_types=(
            [pltpu.VMEM((n_chunks, CW), jnp.int32)]
            + [pltpu.VMEM_SHARED((PT, D), jnp.float32)]
            + [pltpu.VMEM((CW, D), jnp.float32)] * NBUF
            + [pltpu.SemaphoreType.DMA] * (3 * NBUF)
        ),
    )
    def enc(obs_hbm, embed_hbm, pos_hbm, out_hbm, idx_v, pos_sh, *rest):
        bufs = rest[:NBUF]
        ssem = rest[NBUF:2 * NBUF]
        gsem = rest[2 * NBUF:3 * NBUF]
        osem = rest[3 * NBUF:4 * NBUF]
        sid = lax.axis_index("s")
        wid = sid * NC + lax.axis_index("c")
        base = wid * rows_w
        pltpu.sync_copy(obs_hbm.at[wid], idx_v)
        @pl.when(sid == 0)
        def _():
            pltpu.sync_copy(pos_hbm, pos_sh)
        plsc.subcore_barrier()

        def seed(s, b):
            off = (s * CW) % T
            pltpu.async_copy(pos_sh.at[pl.ds(off, CW)], bufs[b], ssem[b])

        seed(0, 0)
        seed(1, 1)

        def visit(s, b):
            pltpu.make_async_copy(pos_sh.at[pl.ds(0, CW)], bufs[b],
                                  ssem[b]).wait()
            d0 = pltpu.async_copy(embed_hbm.at[idx_v.at[s]], bufs[b],
                                  gsem[b], add=True)
            b2 = (b + 2) % NBUF
            @pl.when(s + 2 < n_chunks)
            def _():
                @pl.when(s >= 2)
                def _():
                    pltpu.make_async_copy(
                        bufs[b2],
                        out_hbm.at[pl.ds(base, CW), pl.ds(0, D)],
                        osem[b2]).wait()
                seed(s + 2, b2)
            d0.wait()
            pltpu.async_copy(
                bufs[b],
                out_hbm.at[pl.ds(base + s * CW, CW), pl.ds(0, D)], osem[b])

        def body(g, carry):
            for b in range(NBUF):
                visit(NBUF * g + b, b)
            return carry

        lax.fori_loop(0, n_chunks // NBUF, body, 0)
        for b in range(NBUF):
            pltpu.make_async_copy(
                bufs[b], out_hbm.at[pl.ds(base, CW), pl.ds(0, D)],
                osem[b]).wait()

    return enc


def kernel(obs, embed, pos_embed):
    B, T = obs.shape
    vocab, D = embed.shape
    enc = _build(B, T, D, vocab)
    info = plsc.get_sparse_core_info()
    NW = info.num_cores * info.num_subcores
    obs_i = obs.astype(jnp.int32).reshape(NW, (B * T) // (NW * CW), CW)
    pos_t = pos_embed[:T]
    pos_w = jnp.concatenate([pos_t, pos_t[:CW + 8]], axis=0)
    out = enc(obs_i, embed, pos_w)
    return out[:, :D].reshape(B, T, D)


# speedup vs baseline: 1.9577x; 1.2072x over previous
import functools

import jax
import jax.numpy as jnp
from jax import lax
from jax.experimental import pallas as pl
from jax.experimental.pallas import tpu as pltpu
from jax.experimental.pallas import tpu_sc as plsc

NBUF = 5
CW = 128


def _build(B, T, D, vocab):
    info = plsc.get_sparse_core_info()
    NC, NS = info.num_cores, info.num_subcores
    NW = NC * NS
    BT = B * T
    assert BT % (NW * CW) == 0
    rows_w = BT // NW
    n_chunks = rows_w // CW
    assert rows_w % T == 0
    assert (CW % 8 == 0) and (T % 8 == 0)
    assert n_chunks % NBUF == 0
    PT = T + CW + 8

    mesh = plsc.VectorSubcoreMesh(core_axis_name="c", subcore_axis_name="s")

    @functools.partial(
        pl.kernel,
        out_type=jax.ShapeDtypeStruct((BT, 128), jnp.float32),
        mesh=mesh,
        compiler_params=pltpu.CompilerParams(use_tc_tiling_on_sc=False),
        scratch_types=(
            [pltpu.VMEM((n_chunks, CW), jnp.int32)]
            + [pltpu.VMEM_SHARED((PT, D), jnp.float32)]
            + [pltpu.VMEM((CW, D), jnp.float32)] * NBUF
            + [pltpu.SemaphoreType.DMA] * (3 * NBUF)
        ),
    )
    def enc(obs_hbm, embed_hbm, pos_hbm, out_hbm, idx_v, pos_sh, *rest):
        bufs = rest[:NBUF]
        ssem = rest[NBUF:2 * NBUF]
        gsem = rest[2 * NBUF:3 * NBUF]
        osem = rest[3 * NBUF:4 * NBUF]
        sid = lax.axis_index("s")
        wid = sid * NC + lax.axis_index("c")
        base = wid * rows_w
        pltpu.sync_copy(obs_hbm.at[wid], idx_v)
        @pl.when(sid == 0)
        def _():
            pltpu.sync_copy(pos_hbm, pos_sh)
        plsc.subcore_barrier()

        def seed(s, b):
            off = (s * CW) % T
            pltpu.async_copy(pos_sh.at[pl.ds(off, CW)], bufs[b], ssem[b])

        seed(0, 0)
        seed(1, 1)
        seed(2, 2)

        def gwait(b):
            pltpu.make_async_copy(embed_hbm.at[idx_v.at[0]], bufs[b],
                                  gsem[b]).wait()

        def store(s, b):
            pltpu.async_copy(
                bufs[b],
                out_hbm.at[pl.ds(base + s * CW, CW), pl.ds(0, D)], osem[b])

        def owait(b):
            pltpu.make_async_copy(
                bufs[b], out_hbm.at[pl.ds(base, CW), pl.ds(0, D)],
                osem[b]).wait()

        def visit(s, b):
            pltpu.make_async_copy(pos_sh.at[pl.ds(0, CW)], bufs[b],
                                  ssem[b]).wait()
            pltpu.async_copy(embed_hbm.at[idx_v.at[s]], bufs[b],
                             gsem[b], add=True)
            bp = (b + NBUF - 1) % NBUF
            @pl.when(s >= 1)
            def _():
                gwait(bp)
                store(s - 1, bp)
            b3 = (b + 3) % NBUF
            @pl.when(s + 3 < n_chunks)
            def _():
                @pl.when(s >= 2)
                def _():
                    owait(b3)
                seed(s + 3, b3)

        def body(g, carry):
            for b in range(NBUF):
                visit(NBUF * g + b, b)
            return carry

        lax.fori_loop(0, n_chunks // NBUF, body, 0)
        blast = (n_chunks - 1) % NBUF
        gwait(blast)
        store(n_chunks - 1, blast)
        for b in range(NBUF):
            owait(b)

    return enc


def kernel(obs, embed, pos_embed):
    B, T = obs.shape
    vocab, D = embed.shape
    enc = _build(B, T, D, vocab)
    info = plsc.get_sparse_core_info()
    NW = info.num_cores * info.num_subcores
    obs_i = obs.astype(jnp.int32).reshape(NW, (B * T) // (NW * CW), CW)
    pos_t = pos_embed[:T]
    pos_w = jnp.concatenate([pos_t, pos_t[:CW + 8]], axis=0)
    out = enc(obs_i, embed, pos_w)
    return out[:, :D].reshape(B, T, D)


# speedup vs baseline: 1.9950x; 1.0190x over previous
import functools

import jax
import jax.numpy as jnp
from jax import lax
from jax.experimental import pallas as pl
from jax.experimental.pallas import tpu as pltpu
from jax.experimental.pallas import tpu_sc as plsc

NBUF = 8
CW = 128


def _build(B, T, D, vocab):
    info = plsc.get_sparse_core_info()
    NC, NS = info.num_cores, info.num_subcores
    NW = NC * NS
    BT = B * T
    assert BT % (NW * CW) == 0
    rows_w = BT // NW
    n_chunks = rows_w // CW
    assert rows_w % T == 0
    assert (CW % 8 == 0) and (T % 8 == 0)
    assert n_chunks % NBUF == 0
    PT = T + CW + 8

    mesh = plsc.VectorSubcoreMesh(core_axis_name="c", subcore_axis_name="s")

    @functools.partial(
        pl.kernel,
        out_type=jax.ShapeDtypeStruct((BT, 128), jnp.float32),
        mesh=mesh,
        compiler_params=pltpu.CompilerParams(use_tc_tiling_on_sc=False),
        scratch_types=(
            [pltpu.VMEM((n_chunks, CW), jnp.int32)]
            + [pltpu.VMEM_SHARED((PT, D), jnp.float32)]
            + [pltpu.VMEM((CW, D), jnp.float32)] * NBUF
            + [pltpu.SemaphoreType.DMA] * (3 * NBUF)
        ),
    )
    def enc(obs_hbm, embed_hbm, pos_hbm, out_hbm, idx_v, pos_sh, *rest):
        bufs = rest[:NBUF]
        ssem = rest[NBUF:2 * NBUF]
        gsem = rest[2 * NBUF:3 * NBUF]
        osem = rest[3 * NBUF:4 * NBUF]
        sid = lax.axis_index("s")
        wid = sid * NC + lax.axis_index("c")
        base = wid * rows_w
        pltpu.sync_copy(obs_hbm.at[wid], idx_v)
        @pl.when(sid == 0)
        def _():
            pltpu.sync_copy(pos_hbm, pos_sh)
        plsc.subcore_barrier()

        def seed(s, b):
            off = (s * CW) % T
            pltpu.async_copy(pos_sh.at[pl.ds(off, CW)], bufs[b], ssem[b])

        for p in range(5):
            seed(p, p)

        def gwait(b):
            pltpu.make_async_copy(embed_hbm.at[idx_v.at[0]], bufs[b],
                                  gsem[b]).wait()

        def store(s, b):
            pltpu.async_copy(
                bufs[b],
                out_hbm.at[pl.ds(base + s * CW, CW), pl.ds(0, D)], osem[b])

        def owait(b):
            pltpu.make_async_copy(
                bufs[b], out_hbm.at[pl.ds(base, CW), pl.ds(0, D)],
                osem[b]).wait()

        def visit(s, b):
            pltpu.make_async_copy(pos_sh.at[pl.ds(0, CW)], bufs[b],
                                  ssem[b]).wait()
            pltpu.async_copy(embed_hbm.at[idx_v.at[s]], bufs[b],
                             gsem[b], add=True)
            bp = (b + NBUF - 2) % NBUF
            @pl.when(s >= 2)
            def _():
                gwait(bp)
                store(s - 2, bp)
            b5 = (b + 5) % NBUF
            @pl.when(s + 5 < n_chunks)
            def _():
                @pl.when(s >= 3)
                def _():
                    owait(b5)
                seed(s + 5, b5)

        def body(g, carry):
            for b in range(NBUF):
                visit(NBUF * g + b, b)
            return carry

        lax.fori_loop(0, n_chunks // NBUF, body, 0)
        for s in (n_chunks - 2, n_chunks - 1):
            bl = s % NBUF
            gwait(bl)
            store(s, bl)
        for b in range(NBUF):
            owait(b)

    return enc


def kernel(obs, embed, pos_embed):
    B, T = obs.shape
    vocab, D = embed.shape
    enc = _build(B, T, D, vocab)
    info = plsc.get_sparse_core_info()
    NW = info.num_cores * info.num_subcores
    obs_i = obs.astype(jnp.int32).reshape(NW, (B * T) // (NW * CW), CW)
    pos_t = pos_embed[:T]
    pos_w = jnp.concatenate([pos_t, pos_t[:CW + 8]], axis=0)
    out = enc(obs_i, embed, pos_w)
    return out[:, :D].reshape(B, T, D)
